# baseline (speedup 1.0000x reference)
import functools

import jax
import jax.numpy as jnp
from jax import lax
from jax.experimental import pallas as pl
from jax.experimental.pallas import tpu as pltpu
from jax.experimental.pallas import tpu_sc as plsc

N, D_IN, D_OUT, E, R, B = 10000, 128, 128, 320000, 8, 4

NC, NS = 2, 16
NW = NC * NS
CHUNK = 128
EPW = -(-E // NW)
NCHUNK = -(-EPW // CHUNK)
EPAD = NW * NCHUNK * CHUNK

PASS_SPLITS = (0, 2504, 5008, 7512, N)
NPASS = len(PASS_SPLITS) - 1
ACC_ROWS = 2560
ZROWS_PER_TILE = ACC_ROWS // NS
OROWS_PER_TILE = 152
LIST_LEN = (NCHUNK + 1) * CHUNK
ZBUF_ROWS = 32
PACK = 4096


def _xw_body(att_ref, basis_ref, x_ref, o_ref):
    xb = x_ref[...]
    for r in range(R):
        w = att_ref[r, 0] * basis_ref[0]
        for b in range(1, B):
            w = w + att_ref[r, b] * basis_ref[b]
        o_ref[r] = jnp.dot(xb, w, preferred_element_type=jnp.float32)


def _final_body(p_ref, x_ref, root_ref, bias_ref, o_ref):
    o_ref[...] = (p_ref[0] + p_ref[1]
                  + jnp.dot(x_ref[...], root_ref[...],
                            preferred_element_type=jnp.float32)
                  + bias_ref[...])


def _sc_body(xw_hbm, srcb_hbm, etb_hbm, dstb_hbm, out_hbm,
             sbuf, ebuf, dbuf, plistA, plistB, glist, dlist, rows, rows2,
             zbuf, acc, semA, semB, semSA, semSB):
    c = lax.axis_index("c")
    s = lax.axis_index("s")
    w = s * NC + c

    pltpu.sync_copy(srcb_hbm.at[w], sbuf)
    pltpu.sync_copy(etb_hbm.at[w], ebuf)
    pltpu.sync_copy(dstb_hbm.at[w], dbuf)

    zeros16 = jnp.zeros((16,), jnp.float32)

    def zb(i, carry):
        zbuf[i // 8, pl.ds((i % 8) * 16, 16)] = zeros16
        return carry

    lax.fori_loop(0, ZBUF_ROWS * 8, zb, 0)

    lane = lax.broadcasted_iota(jnp.int32, (16,), 0)
    J_TOTAL = NCHUNK * 4

    def make_cscan(p, plist):
        lo = PASS_SPLITS[p]
        hi = PASS_SPLITS[p + 1]

        def cscan(j, off):
            for i in (2 * j, 2 * j + 1):
                dv = dbuf[i // 8, pl.ds((i % 8) * 16, 16)]
                ev = ebuf[i // 8, pl.ds((i % 8) * 16, 16)]
                sv = sbuf[i // 8, pl.ds((i % 8) * 16, 16)]
                m = (dv >= lo) & (dv < hi)
                pk = (ev * N + sv) * PACK + (dv - lo)
                plsc.store_compressed(plist.at[pl.ds(off, 16)], pk, mask=m)
                off = off + plsc.all_reduce_population_count(m)[0]
            return off

        return cscan

    n_cur = lax.fori_loop(0, J_TOTAL, make_cscan(0, plistA), jnp.int32(0))
    plists = [plistA, plistB]

    for p in range(NPASS):
        lo = PASS_SPLITS[p]
        hi = PASS_SPLITS[p + 1]
        pr = hi - lo
        plist = plists[p % 2]
        plist_nxt = plists[(p + 1) % 2]
        trash = jnp.full((16,), pr, jnp.int32)

        nc = (n_cur + CHUNK - 1) // CHUNK

        k0 = n_cur // 16
        base = k0 * 16
        keep = (base + lane) < n_cur
        plist[pl.ds(base, 16)] = jnp.where(keep, plist[pl.ds(base, 16)], trash)

        def padv(k, carry):
            plist[pl.ds(k * 16, 16)] = trash
            return carry

        lax.fori_loop(k0 + 1, nc * 8, padv, 0)

        def unp(i, carry):
            v = plist[pl.ds(i * 16, 16)]
            glist[i // 8, pl.ds((i % 8) * 16, 16)] = v // PACK
            dlist[i // 8, pl.ds((i % 8) * 16, 16)] = v % PACK
            return carry

        lax.fori_loop(0, nc * 8, unp, 0)

        zbase = s * ZROWS_PER_TILE
        for k in range(ZROWS_PER_TILE // ZBUF_ROWS):
            pltpu.sync_copy(zbuf,
                            acc.at[pl.ds(zbase + k * ZBUF_ROWS, ZBUF_ROWS)])

        plsc.subcore_barrier()

        if p + 1 < NPASS:
            cscan_nxt = make_cscan(p + 1, plist_nxt)
        else:
            cscan_nxt = None

        @pl.when(nc > 0)
        def _prologue():
            pltpu.async_copy(xw_hbm.at[glist.at[0]], rows, semA)

        SS = 12

        def step(r, carry):
            jj, off = carry

            @pl.when(r % 2 == 0)
            def _even():
                pltpu.make_async_copy(
                    xw_hbm.at[glist.at[r]], rows, semA).wait()

                @pl.when(r >= 1)
                def _wb():
                    pltpu.make_async_copy(
                        rows2, acc.at[dlist.at[r]], semSB).wait()

                @pl.when(r + 1 < nc)
                def _pre():
                    pltpu.async_copy(
                        xw_hbm.at[glist.at[r + 1]], rows2, semB)

                pltpu.async_copy(rows, acc.at[dlist.at[r]], semSA, add=True)

            @pl.when(r % 2 == 1)
            def _odd():
                pltpu.make_async_copy(
                    xw_hbm.at[glist.at[r]], rows2, semB).wait()
                pltpu.make_async_copy(
                    rows, acc.at[dlist.at[r]], semSA).wait()

                @pl.when(r + 1 < nc)
                def _pre():
                    pltpu.async_copy(
                        xw_hbm.at[glist.at[r + 1]], rows, semA)

                pltpu.async_copy(rows2, acc.at[dlist.at[r]], semSB, add=True)

            if cscan_nxt is not None:
                hi_j = jnp.minimum(jj + SS, J_TOTAL)
                off = lax.fori_loop(jj, hi_j, cscan_nxt, off)
                jj = hi_j
            return (jj, off)

        jj_end, off_end = lax.fori_loop(0, nc, step,
                                        (jnp.int32(0), jnp.int32(0)))

        @pl.when((nc > 0) & (nc % 2 == 1))
        def _drain_even():
            pltpu.make_async_copy(rows, acc.at[dlist.at[0]], semSA).wait()

        @pl.when(nc % 2 == 0)
        def _drain_odd():
            @pl.when(nc > 0)
            def _dr():
                pltpu.make_async_copy(rows2, acc.at[dlist.at[0]],
                                      semSB).wait()

        if cscan_nxt is not None:
            n_cur = lax.fori_loop(jj_end, J_TOTAL, cscan_nxt, off_end)

        plsc.subcore_barrier()

        ob = s * OROWS_PER_TILE
        pltpu.sync_copy(acc.at[pl.ds(ob, OROWS_PER_TILE)],
                        out_hbm.at[c, pl.ds(lo + ob, OROWS_PER_TILE)])

        if pr > NS * OROWS_PER_TILE:
            @pl.when(s == NS - 1)
            def _copy_tail():
                tb = NS * OROWS_PER_TILE
                pltpu.sync_copy(acc.at[pl.ds(tb, pr - tb)],
                                out_hbm.at[c, pl.ds(lo + tb, pr - tb)])

        if p + 1 < NPASS:
            plsc.subcore_barrier()


_sc_call = functools.partial(
    pl.kernel,
    out_type=jax.ShapeDtypeStruct((NC, N, D_OUT), jnp.float32),
    mesh=plsc.VectorSubcoreMesh(core_axis_name="c", subcore_axis_name="s"),
    compiler_params=pltpu.CompilerParams(needs_layout_passes=False),
    scratch_types=[
        pltpu.VMEM((NCHUNK, CHUNK), jnp.int32),
        pltpu.VMEM((NCHUNK, CHUNK), jnp.int32),
        pltpu.VMEM((NCHUNK, CHUNK), jnp.int32),
        pltpu.VMEM((LIST_LEN,), jnp.int32),
        pltpu.VMEM((LIST_LEN,), jnp.int32),
        pltpu.VMEM((NCHUNK + 1, CHUNK), jnp.int32),
        pltpu.VMEM((NCHUNK + 1, CHUNK), jnp.int32),
        pltpu.VMEM((CHUNK, D_OUT), jnp.float32),
        pltpu.VMEM((CHUNK, D_OUT), jnp.float32),
        pltpu.VMEM((ZBUF_ROWS, D_OUT), jnp.float32),
        pltpu.VMEM_SHARED((ACC_ROWS, D_OUT), jnp.float32),
        pltpu.SemaphoreType.DMA,
        pltpu.SemaphoreType.DMA,
        pltpu.SemaphoreType.DMA,
        pltpu.SemaphoreType.DMA,
    ],
)


@jax.jit
def kernel(x, edge_index, edge_type, basis, att, root, bias):
    blk = 400
    nb = N // blk

    xw = pl.pallas_call(
        _xw_body,
        grid=(nb,),
        in_specs=[
            pl.BlockSpec((R, B), lambda i: (0, 0)),
            pl.BlockSpec((B, D_IN, D_OUT), lambda i: (0, 0, 0)),
            pl.BlockSpec((blk, D_IN), lambda i: (i, 0)),
        ],
        out_specs=pl.BlockSpec((R, blk, D_OUT), lambda i: (0, i, 0)),
        out_shape=jax.ShapeDtypeStruct((R, N, D_OUT), jnp.float32),
    )(att, basis, x)
    xw_flat = xw.reshape(R * N, D_OUT)

    pad = EPAD - E
    src = jnp.pad(edge_index[0].astype(jnp.int32), (0, pad))
    dst = jnp.pad(edge_index[1].astype(jnp.int32), (0, pad),
                  constant_values=N)
    et = jnp.pad(edge_type.astype(jnp.int32), (0, pad))
    srcb = src.reshape(NW, NCHUNK, CHUNK)
    dstb = dst.reshape(NW, NCHUNK, CHUNK)
    etb = et.reshape(NW, NCHUNK, CHUNK)

    partials = _sc_call(_sc_body)(xw_flat, srcb, etb, dstb)

    out = pl.pallas_call(
        _final_body,
        grid=(nb,),
        in_specs=[
            pl.BlockSpec((NC, blk, D_OUT), lambda i: (0, i, 0)),
            pl.BlockSpec((blk, D_IN), lambda i: (i, 0)),
            pl.BlockSpec((D_IN, D_OUT), lambda i: (0, 0)),
            pl.BlockSpec((1, D_OUT), lambda i: (0, 0)),
        ],
        out_specs=pl.BlockSpec((blk, D_OUT), lambda i: (i, 0)),
        out_shape=jax.ShapeDtypeStruct((N, D_OUT), jnp.float32),
    )(partials, x, root, bias.reshape(1, D_OUT))
    return out


# speedup vs baseline: 1.0587x; 1.0587x over previous
import functools

import jax
import jax.numpy as jnp
from jax import lax
from jax.experimental import pallas as pl
from jax.experimental.pallas import tpu as pltpu
from jax.experimental.pallas import tpu_sc as plsc

N, D_IN, D_OUT, E, R, B = 10000, 128, 128, 320000, 8, 4

NC, NS = 2, 16
NW = NC * NS
CHUNK = 128
EPW = -(-E // NW)
NCHUNK = -(-EPW // CHUNK)
EPAD = NW * NCHUNK * CHUNK

PASS_SPLITS = (0, 3336, 6672, N)
NPASS = len(PASS_SPLITS) - 1
ACC_ROWS = 3584
ZROWS_PER_TILE = ACC_ROWS // NS
OROWS_PER_TILE = 208
LIST_LEN = (NCHUNK + 1) * CHUNK
ZBUF_ROWS = 32
PACK = 4096


def _xw_body(att_ref, basis_ref, x_ref, o_ref):
    xb = x_ref[...]
    for r in range(R):
        w = att_ref[r, 0] * basis_ref[0]
        for b in range(1, B):
            w = w + att_ref[r, b] * basis_ref[b]
        o_ref[r] = jnp.dot(xb, w, preferred_element_type=jnp.float32)


def _final_body(p_ref, x_ref, root_ref, bias_ref, o_ref):
    o_ref[...] = (p_ref[0] + p_ref[1]
                  + jnp.dot(x_ref[...], root_ref[...],
                            preferred_element_type=jnp.float32)
                  + bias_ref[...])


def _sc_body(xw_hbm, srcb_hbm, etb_hbm, dstb_hbm, out_hbm,
             sbuf, ebuf, dbuf, plistA, glist, dlist, rows, rows2,
             zbuf, acc, semA, semB, semSA, semSB):
    c = lax.axis_index("c")
    s = lax.axis_index("s")
    w = s * NC + c

    pltpu.sync_copy(srcb_hbm.at[w], sbuf)
    pltpu.sync_copy(etb_hbm.at[w], ebuf)
    pltpu.sync_copy(dstb_hbm.at[w], dbuf)

    zeros16 = jnp.zeros((16,), jnp.float32)

    def zb(i, carry):
        zbuf[i // 8, pl.ds((i % 8) * 16, 16)] = zeros16
        return carry

    lax.fori_loop(0, ZBUF_ROWS * 8, zb, 0)

    lane = lax.broadcasted_iota(jnp.int32, (16,), 0)
    J_TOTAL = NCHUNK * 4

    def make_cscan(p, plist):
        lo = PASS_SPLITS[p]
        hi = PASS_SPLITS[p + 1]

        def cscan(j, off):
            for i in (2 * j, 2 * j + 1):
                dv = dbuf[i // 8, pl.ds((i % 8) * 16, 16)]
                ev = ebuf[i // 8, pl.ds((i % 8) * 16, 16)]
                sv = sbuf[i // 8, pl.ds((i % 8) * 16, 16)]
                m = (dv >= lo) & (dv < hi)
                pk = (ev * N + sv) * PACK + (dv - lo)
                plsc.store_compressed(plist.at[pl.ds(off, 16)], pk, mask=m)
                off = off + plsc.all_reduce_population_count(m)[0]
            return off

        return cscan

    for p in range(NPASS):
        lo = PASS_SPLITS[p]
        hi = PASS_SPLITS[p + 1]
        pr = hi - lo
        plist = plistA
        trash = jnp.full((16,), pr, jnp.int32)

        n_cur = lax.fori_loop(0, J_TOTAL, make_cscan(p, plist), jnp.int32(0))
        nc = (n_cur + CHUNK - 1) // CHUNK

        k0 = n_cur // 16
        base = k0 * 16
        keep = (base + lane) < n_cur
        plist[pl.ds(base, 16)] = jnp.where(keep, plist[pl.ds(base, 16)], trash)

        def padv(k, carry):
            plist[pl.ds(k * 16, 16)] = trash
            return carry

        lax.fori_loop(k0 + 1, nc * 8, padv, 0)

        def unp(i, carry):
            v = plist[pl.ds(i * 16, 16)]
            glist[i // 8, pl.ds((i % 8) * 16, 16)] = v // PACK
            dlist[i // 8, pl.ds((i % 8) * 16, 16)] = v % PACK
            return carry

        lax.fori_loop(0, nc * 8, unp, 0)

        zbase = s * ZROWS_PER_TILE
        for k in range(ZROWS_PER_TILE // ZBUF_ROWS):
            pltpu.sync_copy(zbuf,
                            acc.at[pl.ds(zbase + k * ZBUF_ROWS, ZBUF_ROWS)])

        plsc.subcore_barrier()

        @pl.when(nc > 0)
        def _prologue():
            pltpu.async_copy(xw_hbm.at[glist.at[0]], rows, semA)

        def step(r, carry):
            @pl.when(r % 2 == 0)
            def _even():
                pltpu.make_async_copy(
                    xw_hbm.at[glist.at[r]], rows, semA).wait()

                @pl.when(r >= 1)
                def _wb():
                    pltpu.make_async_copy(
                        rows2, acc.at[dlist.at[r]], semSB).wait()

                @pl.when(r + 1 < nc)
                def _pre():
                    pltpu.async_copy(
                        xw_hbm.at[glist.at[r + 1]], rows2, semB)

                pltpu.async_copy(rows, acc.at[dlist.at[r]], semSA, add=True)

            @pl.when(r % 2 == 1)
            def _odd():
                pltpu.make_async_copy(
                    xw_hbm.at[glist.at[r]], rows2, semB).wait()
                pltpu.make_async_copy(
                    rows, acc.at[dlist.at[r]], semSA).wait()

                @pl.when(r + 1 < nc)
                def _pre():
                    pltpu.async_copy(
                        xw_hbm.at[glist.at[r + 1]], rows, semA)

                pltpu.async_copy(rows2, acc.at[dlist.at[r]], semSB, add=True)

            return carry

        lax.fori_loop(0, nc, step, 0)

        @pl.when((nc > 0) & (nc % 2 == 1))
        def _drain_even():
            pltpu.make_async_copy(rows, acc.at[dlist.at[0]], semSA).wait()

        @pl.when(nc % 2 == 0)
        def _drain_odd():
            @pl.when(nc > 0)
            def _dr():
                pltpu.make_async_copy(rows2, acc.at[dlist.at[0]],
                                      semSB).wait()

        plsc.subcore_barrier()

        ob = s * OROWS_PER_TILE
        pltpu.sync_copy(acc.at[pl.ds(ob, OROWS_PER_TILE)],
                        out_hbm.at[c, pl.ds(lo + ob, OROWS_PER_TILE)])

        if pr > NS * OROWS_PER_TILE:
            @pl.when(s == NS - 1)
            def _copy_tail():
                tb = NS * OROWS_PER_TILE
                pltpu.sync_copy(acc.at[pl.ds(tb, pr - tb)],
                                out_hbm.at[c, pl.ds(lo + tb, pr - tb)])

        if p + 1 < NPASS:
            plsc.subcore_barrier()


_sc_call = functools.partial(
    pl.kernel,
    out_type=jax.ShapeDtypeStruct((NC, N, D_OUT), jnp.float32),
    mesh=plsc.VectorSubcoreMesh(core_axis_name="c", subcore_axis_name="s"),
    compiler_params=pltpu.CompilerParams(needs_layout_passes=False),
    scratch_types=[
        pltpu.VMEM((NCHUNK, CHUNK), jnp.int32),
        pltpu.VMEM((NCHUNK, CHUNK), jnp.int32),
        pltpu.VMEM((NCHUNK, CHUNK), jnp.int32),
        pltpu.VMEM((LIST_LEN,), jnp.int32),
        pltpu.VMEM((NCHUNK + 1, CHUNK), jnp.int32),
        pltpu.VMEM((NCHUNK + 1, CHUNK), jnp.int32),
        pltpu.VMEM((CHUNK, D_OUT), jnp.float32),
        pltpu.VMEM((CHUNK, D_OUT), jnp.float32),
        pltpu.VMEM((ZBUF_ROWS, D_OUT), jnp.float32),
        pltpu.VMEM_SHARED((ACC_ROWS, D_OUT), jnp.float32),
        pltpu.SemaphoreType.DMA,
        pltpu.SemaphoreType.DMA,
        pltpu.SemaphoreType.DMA,
        pltpu.SemaphoreType.DMA,
    ],
)


@jax.jit
def kernel(x, edge_index, edge_type, basis, att, root, bias):
    blk = 400
    nb = N // blk

    xw = pl.pallas_call(
        _xw_body,
        grid=(nb,),
        in_specs=[
            pl.BlockSpec((R, B), lambda i: (0, 0)),
            pl.BlockSpec((B, D_IN, D_OUT), lambda i: (0, 0, 0)),
            pl.BlockSpec((blk, D_IN), lambda i: (i, 0)),
        ],
        out_specs=pl.BlockSpec((R, blk, D_OUT), lambda i: (0, i, 0)),
        out_shape=jax.ShapeDtypeStruct((R, N, D_OUT), jnp.float32),
    )(att, basis, x)
    xw_flat = xw.reshape(R * N, D_OUT)

    pad = EPAD - E
    src = jnp.pad(edge_index[0].astype(jnp.int32), (0, pad))
    dst = jnp.pad(edge_index[1].astype(jnp.int32), (0, pad),
                  constant_values=N)
    et = jnp.pad(edge_type.astype(jnp.int32), (0, pad))
    srcb = src.reshape(NW, NCHUNK, CHUNK)
    dstb = dst.reshape(NW, NCHUNK, CHUNK)
    etb = et.reshape(NW, NCHUNK, CHUNK)

    partials = _sc_call(_sc_body)(xw_flat, srcb, etb, dstb)

    out = pl.pallas_call(
        _final_body,
        grid=(nb,),
        in_specs=[
            pl.BlockSpec((NC, blk, D_OUT), lambda i: (0, i, 0)),
            pl.BlockSpec((blk, D_IN), lambda i: (i, 0)),
            pl.BlockSpec((D_IN, D_OUT), lambda i: (0, 0)),
            pl.BlockSpec((1, D_OUT), lambda i: (0, 0)),
        ],
        out_specs=pl.BlockSpec((blk, D_OUT), lambda i: (i, 0)),
        out_shape=jax.ShapeDtypeStruct((N, D_OUT), jnp.float32),
    )(partials, x, root, bias.reshape(1, D_OUT))
    return out


# speedup vs baseline: 1.0913x; 1.0308x over previous
import functools

import jax
import jax.numpy as jnp
from jax import lax
from jax.experimental import pallas as pl
from jax.experimental.pallas import tpu as pltpu
from jax.experimental.pallas import tpu_sc as plsc

N, D_IN, D_OUT, E, R, B = 10000, 128, 128, 320000, 8, 4

NC, NS = 2, 16
NW = NC * NS
CHUNK = 128
EPW = -(-E // NW)
NCHUNK = -(-EPW // CHUNK)
EPAD = NW * NCHUNK * CHUNK

PASS_SPLITS = (0, 3336, 6672, N)
NPASS = len(PASS_SPLITS) - 1
ACC_ROWS = 3584
ZROWS_PER_TILE = ACC_ROWS // NS
OROWS_PER_TILE = 208
LIST_LEN = (NCHUNK + 1) * CHUNK
ZBUF_ROWS = 32
PACK = 4096


def _xw_body(att_ref, basis_ref, x_ref, o_ref):
    xb = x_ref[...]
    for r in range(R):
        w = att_ref[r, 0] * basis_ref[0]
        for b in range(1, B):
            w = w + att_ref[r, b] * basis_ref[b]
        o_ref[r] = jnp.dot(xb, w, preferred_element_type=jnp.float32)


def _final_body(p_ref, x_ref, root_ref, bias_ref, o_ref):
    o_ref[...] = (p_ref[0] + p_ref[1]
                  + jnp.dot(x_ref[...], root_ref[...],
                            preferred_element_type=jnp.float32)
                  + bias_ref[...])


def _sc_body(xw_hbm, srcb_hbm, etb_hbm, dstb_hbm, out_hbm,
             sbuf, ebuf, dbuf, plistA, glist, dlist, rows, rows2,
             zbuf, acc, semA, semB, semSA, semSB):
    c = lax.axis_index("c")
    s = lax.axis_index("s")
    w = s * NC + c

    pltpu.sync_copy(srcb_hbm.at[w], sbuf)
    pltpu.sync_copy(etb_hbm.at[w], ebuf)
    pltpu.sync_copy(dstb_hbm.at[w], dbuf)

    zeros16 = jnp.zeros((16,), jnp.float32)

    def zb(i, carry):
        zbuf[i // 8, pl.ds((i % 8) * 16, 16)] = zeros16
        return carry

    lax.fori_loop(0, ZBUF_ROWS * 8, zb, 0)

    lane = lax.broadcasted_iota(jnp.int32, (16,), 0)
    J_TOTAL = NCHUNK * 4

    def make_cscan(p, plist):
        lo = PASS_SPLITS[p]
        hi = PASS_SPLITS[p + 1]

        def cscan(j, off):
            for i in (2 * j, 2 * j + 1):
                dv = dbuf[i // 8, pl.ds((i % 8) * 16, 16)]
                ev = ebuf[i // 8, pl.ds((i % 8) * 16, 16)]
                sv = sbuf[i // 8, pl.ds((i % 8) * 16, 16)]
                m = (dv >= lo) & (dv < hi)
                pk = (ev * N + sv) * PACK + (dv - lo)
                plsc.store_compressed(plist.at[pl.ds(off, 16)], pk, mask=m)
                off = off + plsc.all_reduce_population_count(m)[0]
            return off

        return cscan

    for p in range(NPASS):
        lo = PASS_SPLITS[p]
        hi = PASS_SPLITS[p + 1]
        pr = hi - lo
        plist = plistA
        trash = jnp.full((16,), pr, jnp.int32)

        n_cur = lax.fori_loop(0, J_TOTAL, make_cscan(p, plist), jnp.int32(0))
        nc = (n_cur + CHUNK - 1) // CHUNK

        k0 = n_cur // 16
        base = k0 * 16
        keep = (base + lane) < n_cur
        plist[pl.ds(base, 16)] = jnp.where(keep, plist[pl.ds(base, 16)], trash)

        def padv(k, carry):
            plist[pl.ds(k * 16, 16)] = trash
            return carry

        lax.fori_loop(k0 + 1, nc * 8, padv, 0)

        def unp(i, carry):
            v = plist[pl.ds(i * 16, 16)]
            glist[i // 8, pl.ds((i % 8) * 16, 16)] = v // PACK
            dlist[i // 8, pl.ds((i % 8) * 16, 16)] = v % PACK
            return carry

        lax.fori_loop(0, nc * 8, unp, 0)

        zbase = s * ZROWS_PER_TILE
        for k in range(ZROWS_PER_TILE // ZBUF_ROWS):
            pltpu.sync_copy(zbuf,
                            acc.at[pl.ds(zbase + k * ZBUF_ROWS, ZBUF_ROWS)])

        plsc.subcore_barrier()

        @pl.when(nc > 0)
        def _prologue():
            pltpu.async_copy(xw_hbm.at[glist.at[0]], rows, semA)

        def step(r, carry):
            @pl.when(r % 2 == 0)
            def _even():
                pltpu.make_async_copy(
                    xw_hbm.at[glist.at[r]], rows, semA).wait()

                @pl.when(r >= 1)
                def _wb():
                    pltpu.make_async_copy(
                        rows2, acc.at[dlist.at[r]], semSB).wait()

                @pl.when(r + 1 < nc)
                def _pre():
                    pltpu.async_copy(
                        xw_hbm.at[glist.at[r + 1]], rows2, semB)

                pltpu.async_copy(rows, acc.at[dlist.at[r]], semSA, add=True)

            @pl.when(r % 2 == 1)
            def _odd():
                pltpu.make_async_copy(
                    xw_hbm.at[glist.at[r]], rows2, semB).wait()
                pltpu.make_async_copy(
                    rows, acc.at[dlist.at[r]], semSA).wait()

                @pl.when(r + 1 < nc)
                def _pre():
                    pltpu.async_copy(
                        xw_hbm.at[glist.at[r + 1]], rows, semA)

                pltpu.async_copy(rows2, acc.at[dlist.at[r]], semSB, add=True)

            return carry

        lax.fori_loop(0, nc, step, 0)

        @pl.when((nc > 0) & (nc % 2 == 1))
        def _drain_even():
            pltpu.make_async_copy(rows, acc.at[dlist.at[0]], semSA).wait()

        @pl.when(nc % 2 == 0)
        def _drain_odd():
            @pl.when(nc > 0)
            def _dr():
                pltpu.make_async_copy(rows2, acc.at[dlist.at[0]],
                                      semSB).wait()

        plsc.subcore_barrier()

        ob = s * OROWS_PER_TILE
        pltpu.sync_copy(acc.at[pl.ds(ob, OROWS_PER_TILE)],
                        out_hbm.at[c, pl.ds(lo + ob, OROWS_PER_TILE)])

        if pr > NS * OROWS_PER_TILE:
            @pl.when(s == NS - 1)
            def _copy_tail():
                tb = NS * OROWS_PER_TILE
                pltpu.sync_copy(acc.at[pl.ds(tb, pr - tb)],
                                out_hbm.at[c, pl.ds(lo + tb, pr - tb)])

        if p + 1 < NPASS:
            plsc.subcore_barrier()


_sc_call = functools.partial(
    pl.kernel,
    out_type=jax.ShapeDtypeStruct((NC, N, D_OUT), jnp.float32),
    mesh=plsc.VectorSubcoreMesh(core_axis_name="c", subcore_axis_name="s"),
    compiler_params=pltpu.CompilerParams(needs_layout_passes=False),
    scratch_types=[
        pltpu.VMEM((NCHUNK, CHUNK), jnp.int32),
        pltpu.VMEM((NCHUNK, CHUNK), jnp.int32),
        pltpu.VMEM((NCHUNK, CHUNK), jnp.int32),
        pltpu.VMEM((LIST_LEN,), jnp.int32),
        pltpu.VMEM((NCHUNK + 1, CHUNK), jnp.int32),
        pltpu.VMEM((NCHUNK + 1, CHUNK), jnp.int32),
        pltpu.VMEM((CHUNK, D_OUT), jnp.float32),
        pltpu.VMEM((CHUNK, D_OUT), jnp.float32),
        pltpu.VMEM((ZBUF_ROWS, D_OUT), jnp.float32),
        pltpu.VMEM_SHARED((ACC_ROWS, D_OUT), jnp.float32),
        pltpu.SemaphoreType.DMA,
        pltpu.SemaphoreType.DMA,
        pltpu.SemaphoreType.DMA,
        pltpu.SemaphoreType.DMA,
    ],
)


@jax.jit
def kernel(x, edge_index, edge_type, basis, att, root, bias):
    blk = 1000
    nb = N // blk

    xw = pl.pallas_call(
        _xw_body,
        grid=(nb,),
        in_specs=[
            pl.BlockSpec((R, B), lambda i: (0, 0)),
            pl.BlockSpec((B, D_IN, D_OUT), lambda i: (0, 0, 0)),
            pl.BlockSpec((blk, D_IN), lambda i: (i, 0)),
        ],
        out_specs=pl.BlockSpec((R, blk, D_OUT), lambda i: (0, i, 0)),
        out_shape=jax.ShapeDtypeStruct((R, N, D_OUT), jnp.float32),
    )(att, basis, x)
    xw_flat = xw.reshape(R * N, D_OUT)

    pad = EPAD - E
    src = jnp.pad(edge_index[0].astype(jnp.int32), (0, pad))
    dst = jnp.pad(edge_index[1].astype(jnp.int32), (0, pad),
                  constant_values=N)
    et = jnp.pad(edge_type.astype(jnp.int32), (0, pad))
    srcb = src.reshape(NW, NCHUNK, CHUNK)
    dstb = dst.reshape(NW, NCHUNK, CHUNK)
    etb = et.reshape(NW, NCHUNK, CHUNK)

    partials = _sc_call(_sc_body)(xw_flat, srcb, etb, dstb)

    out = pl.pallas_call(
        _final_body,
        grid=(nb,),
        in_specs=[
            pl.BlockSpec((NC, blk, D_OUT), lambda i: (0, i, 0)),
            pl.BlockSpec((blk, D_IN), lambda i: (i, 0)),
            pl.BlockSpec((D_IN, D_OUT), lambda i: (0, 0)),
            pl.BlockSpec((1, D_OUT), lambda i: (0, 0)),
        ],
        out_specs=pl.BlockSpec((blk, D_OUT), lambda i: (i, 0)),
        out_shape=jax.ShapeDtypeStruct((N, D_OUT), jnp.float32),
    )(partials, x, root, bias.reshape(1, D_OUT))
    return out


# speedup vs baseline: 1.1065x; 1.0139x over previous
import functools

import jax
import jax.numpy as jnp
from jax import lax
from jax.experimental import pallas as pl
from jax.experimental.pallas import tpu as pltpu
from jax.experimental.pallas import tpu_sc as plsc

N, D_IN, D_OUT, E, R, B = 10000, 128, 128, 320000, 8, 4

NC, NS = 2, 16
NW = NC * NS
CHUNK = 128
EPW = -(-E // NW)
NCHUNK = -(-EPW // CHUNK)
EPAD = NW * NCHUNK * CHUNK

PASS_SPLITS = (0, 3336, 6672, N)
NPASS = len(PASS_SPLITS) - 1
ACC_ROWS = 3584
ZROWS_PER_TILE = ACC_ROWS // NS
OROWS_PER_TILE = 208
LIST_LEN = (NCHUNK + 1) * CHUNK
ZBUF_ROWS = 32
PACK = 4096


def _xw_body(att_ref, basis_ref, x_ref, o_ref):
    xb = x_ref[...]
    for r in range(R):
        w = att_ref[r, 0] * basis_ref[0]
        for b in range(1, B):
            w = w + att_ref[r, b] * basis_ref[b]
        o_ref[r] = jnp.dot(xb, w, preferred_element_type=jnp.float32)


def _final_body(p_ref, x_ref, root_ref, bias_ref, o_ref):
    o_ref[...] = (p_ref[0] + p_ref[1]
                  + jnp.dot(x_ref[...], root_ref[...],
                            preferred_element_type=jnp.float32)
                  + bias_ref[...])


def _sc_body(xw_hbm, srcb_hbm, etb_hbm, dstb_hbm, out_hbm,
             sbuf, ebuf, dbuf, plistA, glist, dlist, rows, rows2,
             zbuf, acc, semA, semB, semSA, semSB):
    c = lax.axis_index("c")
    s = lax.axis_index("s")
    w = s * NC + c

    pltpu.sync_copy(srcb_hbm.at[w], sbuf)
    pltpu.sync_copy(etb_hbm.at[w], ebuf)
    pltpu.sync_copy(dstb_hbm.at[w], dbuf)

    zeros16 = jnp.zeros((16,), jnp.float32)

    def zb(i, carry):
        zbuf[i // 8, pl.ds((i % 8) * 16, 16)] = zeros16
        return carry

    lax.fori_loop(0, ZBUF_ROWS * 8, zb, 0)

    lane = lax.broadcasted_iota(jnp.int32, (16,), 0)
    J_TOTAL = NCHUNK * 2

    def make_cscan(p, plist):
        lo = PASS_SPLITS[p]
        hi = PASS_SPLITS[p + 1]

        def cscan(j, off):
            for u in range(4):
                i = 4 * j + u
                dv = dbuf[i // 8, pl.ds((i % 8) * 16, 16)]
                ev = ebuf[i // 8, pl.ds((i % 8) * 16, 16)]
                sv = sbuf[i // 8, pl.ds((i % 8) * 16, 16)]
                m = (dv >= lo) & (dv < hi)
                pk = (ev * N + sv) * PACK + (dv - lo)
                plsc.store_compressed(plist.at[pl.ds(off, 16)], pk, mask=m)
                off = off + plsc.all_reduce_population_count(m)[0]
            return off

        return cscan

    for p in range(NPASS):
        lo = PASS_SPLITS[p]
        hi = PASS_SPLITS[p + 1]
        pr = hi - lo
        plist = plistA
        trash = jnp.full((16,), pr, jnp.int32)

        n_cur = lax.fori_loop(0, J_TOTAL, make_cscan(p, plist), jnp.int32(0))
        nc = (n_cur + CHUNK - 1) // CHUNK

        k0 = n_cur // 16
        base = k0 * 16
        keep = (base + lane) < n_cur
        plist[pl.ds(base, 16)] = jnp.where(keep, plist[pl.ds(base, 16)], trash)

        def padv(k, carry):
            plist[pl.ds(k * 16, 16)] = trash
            return carry

        lax.fori_loop(k0 + 1, nc * 8, padv, 0)

        def unp(i, carry):
            v = plist[pl.ds(i * 16, 16)]
            glist[i // 8, pl.ds((i % 8) * 16, 16)] = v // PACK
            dlist[i // 8, pl.ds((i % 8) * 16, 16)] = v % PACK
            return carry

        lax.fori_loop(0, nc * 8, unp, 0)

        zbase = s * ZROWS_PER_TILE
        for k in range(ZROWS_PER_TILE // ZBUF_ROWS):
            pltpu.sync_copy(zbuf,
                            acc.at[pl.ds(zbase + k * ZBUF_ROWS, ZBUF_ROWS)])

        plsc.subcore_barrier()

        @pl.when(nc > 0)
        def _prologue():
            pltpu.async_copy(xw_hbm.at[glist.at[0]], rows, semA)

        def step(r, carry):
            @pl.when(r % 2 == 0)
            def _even():
                pltpu.make_async_copy(
                    xw_hbm.at[glist.at[r]], rows, semA).wait()

                @pl.when(r >= 1)
                def _wb():
                    pltpu.make_async_copy(
                        rows2, acc.at[dlist.at[r]], semSB).wait()

                @pl.when(r + 1 < nc)
                def _pre():
                    pltpu.async_copy(
                        xw_hbm.at[glist.at[r + 1]], rows2, semB)

                pltpu.async_copy(rows, acc.at[dlist.at[r]], semSA, add=True)

            @pl.when(r % 2 == 1)
            def _odd():
                pltpu.make_async_copy(
                    xw_hbm.at[glist.at[r]], rows2, semB).wait()
                pltpu.make_async_copy(
                    rows, acc.at[dlist.at[r]], semSA).wait()

                @pl.when(r + 1 < nc)
                def _pre():
                    pltpu.async_copy(
                        xw_hbm.at[glist.at[r + 1]], rows, semA)

                pltpu.async_copy(rows2, acc.at[dlist.at[r]], semSB, add=True)

            return carry

        lax.fori_loop(0, nc, step, 0)

        @pl.when((nc > 0) & (nc % 2 == 1))
        def _drain_even():
            pltpu.make_async_copy(rows, acc.at[dlist.at[0]], semSA).wait()

        @pl.when(nc % 2 == 0)
        def _drain_odd():
            @pl.when(nc > 0)
            def _dr():
                pltpu.make_async_copy(rows2, acc.at[dlist.at[0]],
                                      semSB).wait()

        plsc.subcore_barrier()

        ob = s * OROWS_PER_TILE
        pltpu.sync_copy(acc.at[pl.ds(ob, OROWS_PER_TILE)],
                        out_hbm.at[c, pl.ds(lo + ob, OROWS_PER_TILE)])

        if pr > NS * OROWS_PER_TILE:
            @pl.when(s == NS - 1)
            def _copy_tail():
                tb = NS * OROWS_PER_TILE
                pltpu.sync_copy(acc.at[pl.ds(tb, pr - tb)],
                                out_hbm.at[c, pl.ds(lo + tb, pr - tb)])

        if p + 1 < NPASS:
            plsc.subcore_barrier()


_sc_call = functools.partial(
    pl.kernel,
    out_type=jax.ShapeDtypeStruct((NC, N, D_OUT), jnp.float32),
    mesh=plsc.VectorSubcoreMesh(core_axis_name="c", subcore_axis_name="s"),
    compiler_params=pltpu.CompilerParams(needs_layout_passes=False),
    scratch_types=[
        pltpu.VMEM((NCHUNK, CHUNK), jnp.int32),
        pltpu.VMEM((NCHUNK, CHUNK), jnp.int32),
        pltpu.VMEM((NCHUNK, CHUNK), jnp.int32),
        pltpu.VMEM((LIST_LEN,), jnp.int32),
        pltpu.VMEM((NCHUNK + 1, CHUNK), jnp.int32),
        pltpu.VMEM((NCHUNK + 1, CHUNK), jnp.int32),
        pltpu.VMEM((CHUNK, D_OUT), jnp.float32),
        pltpu.VMEM((CHUNK, D_OUT), jnp.float32),
        pltpu.VMEM((ZBUF_ROWS, D_OUT), jnp.float32),
        pltpu.VMEM_SHARED((ACC_ROWS, D_OUT), jnp.float32),
        pltpu.SemaphoreType.DMA,
        pltpu.SemaphoreType.DMA,
        pltpu.SemaphoreType.DMA,
        pltpu.SemaphoreType.DMA,
    ],
)


@jax.jit
def kernel(x, edge_index, edge_type, basis, att, root, bias):
    blk = 2000
    nb = N // blk

    xw = pl.pallas_call(
        _xw_body,
        grid=(nb,),
        in_specs=[
            pl.BlockSpec((R, B), lambda i: (0, 0)),
            pl.BlockSpec((B, D_IN, D_OUT), lambda i: (0, 0, 0)),
            pl.BlockSpec((blk, D_IN), lambda i: (i, 0)),
        ],
        out_specs=pl.BlockSpec((R, blk, D_OUT), lambda i: (0, i, 0)),
        out_shape=jax.ShapeDtypeStruct((R, N, D_OUT), jnp.float32),
    )(att, basis, x)
    xw_flat = xw.reshape(R * N, D_OUT)

    pad = EPAD - E
    src = jnp.pad(edge_index[0].astype(jnp.int32), (0, pad))
    dst = jnp.pad(edge_index[1].astype(jnp.int32), (0, pad),
                  constant_values=N)
    et = jnp.pad(edge_type.astype(jnp.int32), (0, pad))
    srcb = src.reshape(NW, NCHUNK, CHUNK)
    dstb = dst.reshape(NW, NCHUNK, CHUNK)
    etb = et.reshape(NW, NCHUNK, CHUNK)

    partials = _sc_call(_sc_body)(xw_flat, srcb, etb, dstb)

    out = pl.pallas_call(
        _final_body,
        grid=(nb,),
        in_specs=[
            pl.BlockSpec((NC, blk, D_OUT), lambda i: (0, i, 0)),
            pl.BlockSpec((blk, D_IN), lambda i: (i, 0)),
            pl.BlockSpec((D_IN, D_OUT), lambda i: (0, 0)),
            pl.BlockSpec((1, D_OUT), lambda i: (0, 0)),
        ],
        out_specs=pl.BlockSpec((blk, D_OUT), lambda i: (i, 0)),
        out_shape=jax.ShapeDtypeStruct((N, D_OUT), jnp.float32),
    )(partials, x, root, bias.reshape(1, D_OUT))
    return out


# speedup vs baseline: 1.4083x; 1.2727x over previous
import functools

import jax
import jax.numpy as jnp
from jax import lax
from jax.experimental import pallas as pl
from jax.experimental.pallas import tpu as pltpu
from jax.experimental.pallas import tpu_sc as plsc

N, D_IN, D_OUT, E, R, B = 10000, 128, 128, 320000, 8, 4

NC, NS = 2, 16
NW = NC * NS
CHUNK = 128
EPW = -(-E // NW)
NCHUNK = -(-EPW // CHUNK)
EPAD = NW * NCHUNK * CHUNK

PASS_SPLITS = (0, 5000, N)
NPASS = len(PASS_SPLITS) - 1
ACC_ROWS = 5120
ZROWS_PER_TILE = ACC_ROWS // NS
OROWS_PER_TILE = 312
LIST_LEN = (NCHUNK + 1) * CHUNK
ZBUF_ROWS = 16
PACK = 8192


def _xw_body(att_ref, basis_ref, x_ref, o_ref):
    xb = x_ref[...]
    for r in range(R):
        w = att_ref[r, 0] * basis_ref[0]
        for b in range(1, B):
            w = w + att_ref[r, b] * basis_ref[b]
        o_ref[r] = jnp.dot(xb, w, preferred_element_type=jnp.float32)


def _final_body(p_ref, x_ref, root_ref, bias_ref, o_ref):
    o_ref[...] = (p_ref[0] + p_ref[1]
                  + jnp.dot(x_ref[...], root_ref[...],
                            preferred_element_type=jnp.float32)
                  + bias_ref[...])


def _sc_body(xw_hbm, srcb_hbm, etb_hbm, dstb_hbm, out_hbm,
             sbuf, ebuf, dbuf, plistA, dlist, rows, rows2,
             zbuf, acc, semA, semB, semSA, semSB):
    c = lax.axis_index("c")
    s = lax.axis_index("s")
    w = s * NC + c

    pltpu.sync_copy(srcb_hbm.at[w], sbuf)
    pltpu.sync_copy(etb_hbm.at[w], ebuf)
    pltpu.sync_copy(dstb_hbm.at[w], dbuf)

    zeros16 = jnp.zeros((16,), jnp.float32)

    def zb(i, carry):
        zbuf[i // 8, pl.ds((i % 8) * 16, 16)] = zeros16
        return carry

    lax.fori_loop(0, ZBUF_ROWS * 8, zb, 0)

    lane = lax.broadcasted_iota(jnp.int32, (16,), 0)
    J_TOTAL = NCHUNK * 2

    def make_cscan(p, plist):
        lo = PASS_SPLITS[p]
        hi = PASS_SPLITS[p + 1]

        def cscan(j, off):
            for u in range(4):
                i = 4 * j + u
                dv = dbuf[i // 8, pl.ds((i % 8) * 16, 16)]
                ev = ebuf[i // 8, pl.ds((i % 8) * 16, 16)]
                sv = sbuf[i // 8, pl.ds((i % 8) * 16, 16)]
                m = (dv >= lo) & (dv < hi)
                pk = (ev * N + sv) * PACK + (dv - lo)
                plsc.store_compressed(plist.at[pl.ds(off, 16)], pk, mask=m)
                off = off + plsc.all_reduce_population_count(m)[0]
            return off

        return cscan

    for p in range(NPASS):
        lo = PASS_SPLITS[p]
        hi = PASS_SPLITS[p + 1]
        pr = hi - lo
        plist = plistA
        trash = jnp.full((16,), pr, jnp.int32)

        n_cur = lax.fori_loop(0, J_TOTAL, make_cscan(p, plist), jnp.int32(0))
        nc = (n_cur + CHUNK - 1) // CHUNK

        k0 = n_cur // 16
        base = k0 * 16
        keep = (base + lane) < n_cur
        plist[pl.ds(base, 16)] = jnp.where(keep, plist[pl.ds(base, 16)], trash)

        def padv(k, carry):
            plist[pl.ds(k * 16, 16)] = trash
            return carry

        lax.fori_loop(k0 + 1, nc * 8, padv, 0)

        def unp(i, carry):
            v = plist[pl.ds(i * 16, 16)]
            dlist[i // 8, pl.ds((i % 8) * 16, 16)] = v % PACK
            plist[pl.ds(i * 16, 16)] = v // PACK
            return carry

        lax.fori_loop(0, nc * 8, unp, 0)

        zbase = s * ZROWS_PER_TILE
        for k in range(ZROWS_PER_TILE // ZBUF_ROWS):
            pltpu.sync_copy(zbuf,
                            acc.at[pl.ds(zbase + k * ZBUF_ROWS, ZBUF_ROWS)])

        plsc.subcore_barrier()

        @pl.when(nc > 0)
        def _prologue():
            pltpu.async_copy(xw_hbm.at[plist.at[pl.ds(0, CHUNK)]], rows, semA)

        def step(r, carry):
            @pl.when(r % 2 == 0)
            def _even():
                pltpu.make_async_copy(
                    xw_hbm.at[plist.at[pl.ds(r * CHUNK, CHUNK)]], rows,
                    semA).wait()

                @pl.when(r >= 1)
                def _wb():
                    pltpu.make_async_copy(
                        rows2, acc.at[dlist.at[r]], semSB).wait()

                @pl.when(r + 1 < nc)
                def _pre():
                    pltpu.async_copy(
                        xw_hbm.at[plist.at[pl.ds((r + 1) * CHUNK, CHUNK)]],
                        rows2, semB)

                pltpu.async_copy(rows, acc.at[dlist.at[r]], semSA, add=True)

            @pl.when(r % 2 == 1)
            def _odd():
                pltpu.make_async_copy(
                    xw_hbm.at[plist.at[pl.ds(r * CHUNK, CHUNK)]], rows2,
                    semB).wait()
                pltpu.make_async_copy(
                    rows, acc.at[dlist.at[r]], semSA).wait()

                @pl.when(r + 1 < nc)
                def _pre():
                    pltpu.async_copy(
                        xw_hbm.at[plist.at[pl.ds((r + 1) * CHUNK, CHUNK)]],
                        rows, semA)

                pltpu.async_copy(rows2, acc.at[dlist.at[r]], semSB, add=True)

            return carry

        lax.fori_loop(0, nc, step, 0)

        @pl.when((nc > 0) & (nc % 2 == 1))
        def _drain_even():
            pltpu.make_async_copy(rows, acc.at[dlist.at[0]], semSA).wait()

        @pl.when(nc % 2 == 0)
        def _drain_odd():
            @pl.when(nc > 0)
            def _dr():
                pltpu.make_async_copy(rows2, acc.at[dlist.at[0]],
                                      semSB).wait()

        plsc.subcore_barrier()

        ob = s * OROWS_PER_TILE
        pltpu.sync_copy(acc.at[pl.ds(ob, OROWS_PER_TILE)],
                        out_hbm.at[c, pl.ds(lo + ob, OROWS_PER_TILE)])

        if pr > NS * OROWS_PER_TILE:
            @pl.when(s == NS - 1)
            def _copy_tail():
                tb = NS * OROWS_PER_TILE
                pltpu.sync_copy(acc.at[pl.ds(tb, pr - tb)],
                                out_hbm.at[c, pl.ds(lo + tb, pr - tb)])

        if p + 1 < NPASS:
            plsc.subcore_barrier()


_sc_call = functools.partial(
    pl.kernel,
    out_type=jax.ShapeDtypeStruct((NC, N, D_OUT), jnp.float32),
    mesh=plsc.VectorSubcoreMesh(core_axis_name="c", subcore_axis_name="s"),
    compiler_params=pltpu.CompilerParams(needs_layout_passes=False),
    scratch_types=[
        pltpu.VMEM((NCHUNK, CHUNK), jnp.int32),
        pltpu.VMEM((NCHUNK, CHUNK), jnp.int32),
        pltpu.VMEM((NCHUNK, CHUNK), jnp.int32),
        pltpu.VMEM((LIST_LEN,), jnp.int32),
        pltpu.VMEM((NCHUNK + 1, CHUNK), jnp.int32),
        pltpu.VMEM((CHUNK, D_OUT), jnp.float32),
        pltpu.VMEM((CHUNK, D_OUT), jnp.float32),
        pltpu.VMEM((ZBUF_ROWS, D_OUT), jnp.float32),
        pltpu.VMEM_SHARED((ACC_ROWS, D_OUT), jnp.float32),
        pltpu.SemaphoreType.DMA,
        pltpu.SemaphoreType.DMA,
        pltpu.SemaphoreType.DMA,
        pltpu.SemaphoreType.DMA,
    ],
)


@jax.jit
def kernel(x, edge_index, edge_type, basis, att, root, bias):
    blk = 2000
    nb = N // blk

    xw = pl.pallas_call(
        _xw_body,
        grid=(nb,),
        in_specs=[
            pl.BlockSpec((R, B), lambda i: (0, 0)),
            pl.BlockSpec((B, D_IN, D_OUT), lambda i: (0, 0, 0)),
            pl.BlockSpec((blk, D_IN), lambda i: (i, 0)),
        ],
        out_specs=pl.BlockSpec((R, blk, D_OUT), lambda i: (0, i, 0)),
        out_shape=jax.ShapeDtypeStruct((R, N, D_OUT), jnp.float32),
    )(att, basis, x)
    xw_flat = xw.reshape(R * N, D_OUT)

    pad = EPAD - E
    src = jnp.pad(edge_index[0].astype(jnp.int32), (0, pad))
    dst = jnp.pad(edge_index[1].astype(jnp.int32), (0, pad),
                  constant_values=N)
    et = jnp.pad(edge_type.astype(jnp.int32), (0, pad))
    srcb = src.reshape(NW, NCHUNK, CHUNK)
    dstb = dst.reshape(NW, NCHUNK, CHUNK)
    etb = et.reshape(NW, NCHUNK, CHUNK)

    partials = _sc_call(_sc_body)(xw_flat, srcb, etb, dstb)

    out = pl.pallas_call(
        _final_body,
        grid=(nb,),
        in_specs=[
            pl.BlockSpec((NC, blk, D_OUT), lambda i: (0, i, 0)),
            pl.BlockSpec((blk, D_IN), lambda i: (i, 0)),
            pl.BlockSpec((D_IN, D_OUT), lambda i: (0, 0)),
            pl.BlockSpec((1, D_OUT), lambda i: (0, 0)),
        ],
        out_specs=pl.BlockSpec((blk, D_OUT), lambda i: (i, 0)),
        out_shape=jax.ShapeDtypeStruct((N, D_OUT), jnp.float32),
    )(partials, x, root, bias.reshape(1, D_OUT))
    return out
